# baseline (speedup 1.0000x reference)
import functools

import jax
import jax.numpy as jnp
from jax import lax
from jax.experimental import pallas as pl
from jax.experimental.pallas import tpu as pltpu
from jax.experimental.pallas import tpu_sc as plsc

B, N, M, K = 2, 4096, 4096, 16
D, H = 256, 8
C = D // H

NC, NS = 2, 16
NW = NC * NS
NIDX = B * N * K
CHUNKS = (16384, 49152, 49152, 16384)
CH = 128
QB = 256


def _ln(x, g, b, eps=1e-5):
    mu = jnp.mean(x, axis=-1, keepdims=True)
    var = jnp.mean((x - mu) ** 2, axis=-1, keepdims=True)
    return (x - mu) * jax.lax.rsqrt(var + eps) * g + b


def _kv_body(m_ref, wkv_ref, bkv_ref, kv_ref):
    m = m_ref[...]
    kv = jnp.dot(m, wkv_ref[...], preferred_element_type=jnp.float32) + bkv_ref[...]
    kbits = jax.lax.bitcast_convert_type(kv[:, :D], jnp.int32)
    vbits = jax.lax.bitcast_convert_type(kv[:, D:], jnp.int32)
    rne = lambda u: u + 0x7FFF + ((u >> 16) & 1)
    kv_ref[...] = (rne(vbits) & jnp.int32(-65536)) | ((rne(kbits) >> 16) & 0xFFFF)


def _proj_kv(m2d, wkv_t, bkv2, b):
    rb = 1024
    nb = M // rb
    return pl.pallas_call(
        _kv_body,
        grid=(nb,),
        in_specs=[
            pl.BlockSpec((rb, D), lambda i: (b * nb + i, 0)),
            pl.BlockSpec((D, 2 * D), lambda i: (0, 0)),
            pl.BlockSpec((1, 2 * D), lambda i: (0, 0)),
        ],
        out_specs=pl.BlockSpec((rb, D), lambda i: (i, 0)),
        out_shape=jax.ShapeDtypeStruct((M, D), jnp.int32),
    )(m2d, wkv_t, bkv2)


def _sc_gather(idx_chunk, kv_b, nidx_p):
    mesh = plsc.VectorSubcoreMesh(
        core_axis_name="c", subcore_axis_name="s", num_cores=NC, num_subcores=NS
    )
    idx_per_w = nidx_p // NW

    @functools.partial(
        pl.kernel,
        out_type=jax.ShapeDtypeStruct((nidx_p, D), jnp.int32),
        mesh=mesh,
        scratch_types=[
            pltpu.VMEM((idx_per_w,), jnp.int32),
            pltpu.VMEM((CH, D), jnp.int32),
            pltpu.VMEM((CH, D), jnp.int32),
            pltpu.SemaphoreType.DMA,
            pltpu.SemaphoreType.DMA,
        ],
    )
    def gather_kernel(idx_hbm, kv_hbm, out_hbm, idx_v, rows_a, rows_b, sem_a, sem_b):
        cid = lax.axis_index("c")
        sid = lax.axis_index("s")
        wid = sid * NC + cid
        base = wid * idx_per_w
        pltpu.sync_copy(idx_hbm.at[pl.ds(base, idx_per_w)], idx_v)

        nsteps = idx_per_w // CH

        pltpu.async_copy(kv_hbm.at[idx_v.at[pl.ds(0, CH)]], rows_a, sem_a)

        @pl.loop(0, nsteps, step=2)
        def _(s):
            pltpu.async_copy(
                kv_hbm.at[idx_v.at[pl.ds((s + 1) * CH, CH)]], rows_b, sem_b
            )
            pltpu.make_async_copy(
                kv_hbm.at[idx_v.at[pl.ds(s * CH, CH)]], rows_a, sem_a
            ).wait()
            pltpu.sync_copy(rows_a, out_hbm.at[pl.ds(base + s * CH, CH)])

            @pl.when(s + 2 < nsteps)
            def _():
                pltpu.async_copy(
                    kv_hbm.at[idx_v.at[pl.ds((s + 2) * CH, CH)]], rows_a, sem_a
                )

            pltpu.make_async_copy(
                kv_hbm.at[idx_v.at[pl.ds((s + 1) * CH, CH)]], rows_b, sem_b
            ).wait()
            pltpu.sync_copy(rows_b, out_hbm.at[pl.ds(base + (s + 1) * CH, CH)])

    return gather_kernel(idx_chunk, kv_b)


def _attn_body(kv_ref, x_ref, wq_ref, bq_ref, sum_ref, exp_ref, wl_ref, bl_ref,
               g1_ref, bn1_ref, we_ref, be_ref, ws_ref, bs_ref, g2_ref, bn2_ref,
               out_ref):
    qb = x_ref.shape[0]
    xin = x_ref[...]
    q = jnp.dot(xin, wq_ref[...], preferred_element_type=jnp.float32) + bq_ref[...]
    kvi = kv_ref[...]
    k = jax.lax.bitcast_convert_type(kvi << 16, jnp.float32)
    v = jax.lax.bitcast_convert_type(kvi & jnp.int32(-65536), jnp.float32)
    qrep = jnp.broadcast_to(q.reshape(qb, 1, D), (qb, K, D)).reshape(qb * K, D)
    prod = k * qrep
    scores = jnp.dot(prod, sum_ref[...],
                     preferred_element_type=jnp.float32) * (C ** -0.5)
    scores = scores.reshape(qb, K, H)
    mx = jnp.max(scores, axis=1, keepdims=True)
    e = jnp.exp(scores - mx)
    attn = (e / jnp.sum(e, axis=1, keepdims=True)).reshape(qb * K, H)
    attn_exp = jnp.dot(attn, exp_ref[...], preferred_element_type=jnp.float32)
    hid = jnp.sum((attn_exp * v).reshape(qb, K, D), axis=1)
    hid = jnp.dot(hid, wl_ref[...], preferred_element_type=jnp.float32) + bl_ref[...]
    x = _ln(hid + xin, g1_ref[...], bn1_ref[...])
    h = jnp.maximum(
        jnp.dot(x, we_ref[...], preferred_element_type=jnp.float32) + be_ref[...], 0.0
    )
    h = jnp.dot(h, ws_ref[...], preferred_element_type=jnp.float32) + bs_ref[...]
    out_ref[...] = _ln(x + h, g2_ref[...], bn2_ref[...])


def _attn_post(qstart, nq, kvg, x2d, wq_t, bq2, sum_mat, exp_mat,
               wl_t, bl2, g12, bn12, we_t, be2, ws_t, bs2, g22, bn22):
    base = qstart // QB
    grid = (nq // QB,)
    wspec = lambda shape: pl.BlockSpec(shape, lambda i: (0, 0))
    return pl.pallas_call(
        _attn_body,
        grid=grid,
        in_specs=[
            pl.BlockSpec((QB * K, D), lambda i: (i, 0)),
            pl.BlockSpec((QB, D), lambda i: (base + i, 0)),
            wspec((D, D)), wspec((1, D)),
            wspec((D, H)), wspec((H, D)),
            wspec((D, D)), wspec((1, D)), wspec((1, D)), wspec((1, D)),
            wspec((D, 2 * D)), wspec((1, 2 * D)),
            wspec((2 * D, D)), wspec((1, D)), wspec((1, D)), wspec((1, D)),
        ],
        out_specs=pl.BlockSpec((QB, D), lambda i: (i, 0)),
        out_shape=jax.ShapeDtypeStruct((nq, D), jnp.float32),
    )(kvg, x2d, wq_t, bq2, sum_mat, exp_mat,
      wl_t, bl2, g12, bn12, we_t, be2, ws_t, bs2, g22, bn22)


def kernel(input_states, memory_states, indices, Wq, bq, Wk, bk, Wv, bv,
           Wl, bl, g1, bn1, We, be, Ws, bs, g2, bn2):
    x2d = input_states.reshape(B * N, D)
    m2d = memory_states.reshape(B * M, D)
    wkv_t = jnp.concatenate([Wk.T, Wv.T], axis=1)
    bkv2 = jnp.concatenate([bk, bv]).reshape(1, 2 * D)

    head = jnp.arange(D, dtype=jnp.int32) // C
    sum_mat = (head[:, None] == jnp.arange(H, dtype=jnp.int32)[None, :]
               ).astype(jnp.float32)
    exp_mat = sum_mat.T

    kv_tables = [_proj_kv(m2d, wkv_t, bkv2, b) for b in range(B)]
    idx_flat = indices.reshape(NIDX)
    post_args = (
        Wq.T, bq.reshape(1, D), sum_mat, exp_mat,
        Wl.T, bl.reshape(1, D), g1.reshape(1, D), bn1.reshape(1, D),
        We.T, be.reshape(1, 2 * D), Ws.T, bs.reshape(1, D),
        g2.reshape(1, D), bn2.reshape(1, D),
    )
    outs = []
    off = 0
    for nidx_p in CHUNKS:
        idx_c = lax.slice_in_dim(idx_flat, off, off + nidx_p)
        kvg = _sc_gather(idx_c, kv_tables[off // (N * K)], nidx_p)
        outs.append(_attn_post(off // K, nidx_p // K, kvg, x2d, *post_args))
        off += nidx_p
    return jnp.concatenate(outs, axis=0).reshape(B, N, D)


# speedup vs baseline: 1.0975x; 1.0975x over previous
import functools

import jax
import jax.numpy as jnp
from jax import lax
from jax.experimental import pallas as pl
from jax.experimental.pallas import tpu as pltpu
from jax.experimental.pallas import tpu_sc as plsc

B, N, M, K = 2, 4096, 4096, 16
D, H = 256, 8
C = D // H

NC, NS = 2, 16
NW = NC * NS
NIDX = B * N * K
CHUNKS = (32768, 32768, 32768, 32768)
CH = 128
QB = 512


def _ln(x, g, b, eps=1e-5):
    mu = jnp.mean(x, axis=-1, keepdims=True)
    var = jnp.mean((x - mu) ** 2, axis=-1, keepdims=True)
    return (x - mu) * jax.lax.rsqrt(var + eps) * g + b


def _kv_body(m_ref, wkv_ref, bkv_ref, kv_ref):
    m = m_ref[...]
    kv = jnp.dot(m, wkv_ref[...], preferred_element_type=jnp.float32) + bkv_ref[...]
    kbits = jax.lax.bitcast_convert_type(kv[:, :D], jnp.int32)
    vbits = jax.lax.bitcast_convert_type(kv[:, D:], jnp.int32)
    rne = lambda u: u + 0x7FFF + ((u >> 16) & 1)
    kv_ref[...] = (rne(vbits) & jnp.int32(-65536)) | ((rne(kbits) >> 16) & 0xFFFF)


def _proj_kv(m2d, wkv_t, bkv2, b):
    rb = 1024
    nb = M // rb
    return pl.pallas_call(
        _kv_body,
        grid=(nb,),
        in_specs=[
            pl.BlockSpec((rb, D), lambda i: (b * nb + i, 0)),
            pl.BlockSpec((D, 2 * D), lambda i: (0, 0)),
            pl.BlockSpec((1, 2 * D), lambda i: (0, 0)),
        ],
        out_specs=pl.BlockSpec((rb, D), lambda i: (i, 0)),
        out_shape=jax.ShapeDtypeStruct((M, D), jnp.int32),
    )(m2d, wkv_t, bkv2)


def _sc_gather(idx_chunk, kv_b, nidx_p):
    mesh = plsc.VectorSubcoreMesh(
        core_axis_name="c", subcore_axis_name="s", num_cores=NC, num_subcores=NS
    )
    idx_per_w = nidx_p // NW

    @functools.partial(
        pl.kernel,
        out_type=jax.ShapeDtypeStruct((nidx_p, D), jnp.int32),
        mesh=mesh,
        scratch_types=[
            pltpu.VMEM((idx_per_w,), jnp.int32),
            pltpu.VMEM((CH, D), jnp.int32),
            pltpu.VMEM((CH, D), jnp.int32),
            pltpu.SemaphoreType.DMA,
            pltpu.SemaphoreType.DMA,
        ],
    )
    def gather_kernel(idx_hbm, kv_hbm, out_hbm, idx_v, rows_a, rows_b, sem_a, sem_b):
        cid = lax.axis_index("c")
        sid = lax.axis_index("s")
        wid = sid * NC + cid
        base = wid * idx_per_w
        pltpu.sync_copy(idx_hbm.at[pl.ds(base, idx_per_w)], idx_v)

        nsteps = idx_per_w // CH

        pltpu.async_copy(kv_hbm.at[idx_v.at[pl.ds(0, CH)]], rows_a, sem_a)

        @pl.loop(0, nsteps, step=2)
        def _(s):
            pltpu.async_copy(
                kv_hbm.at[idx_v.at[pl.ds((s + 1) * CH, CH)]], rows_b, sem_b
            )
            pltpu.make_async_copy(
                kv_hbm.at[idx_v.at[pl.ds(s * CH, CH)]], rows_a, sem_a
            ).wait()
            pltpu.sync_copy(rows_a, out_hbm.at[pl.ds(base + s * CH, CH)])

            @pl.when(s + 2 < nsteps)
            def _():
                pltpu.async_copy(
                    kv_hbm.at[idx_v.at[pl.ds((s + 2) * CH, CH)]], rows_a, sem_a
                )

            pltpu.make_async_copy(
                kv_hbm.at[idx_v.at[pl.ds((s + 1) * CH, CH)]], rows_b, sem_b
            ).wait()
            pltpu.sync_copy(rows_b, out_hbm.at[pl.ds(base + (s + 1) * CH, CH)])

    return gather_kernel(idx_chunk, kv_b)


def _attn_body(kv_ref, x_ref, wq_ref, bq_ref, sum_ref, exp_ref, wl_ref, bl_ref,
               g1_ref, bn1_ref, we_ref, be_ref, ws_ref, bs_ref, g2_ref, bn2_ref,
               out_ref):
    qb = x_ref.shape[0]
    xin = x_ref[...]
    q = jnp.dot(xin, wq_ref[...], preferred_element_type=jnp.float32) + bq_ref[...]
    kvi = kv_ref[...]
    k = jax.lax.bitcast_convert_type(kvi << 16, jnp.float32)
    v = jax.lax.bitcast_convert_type(kvi & jnp.int32(-65536), jnp.float32)
    qrep = jnp.broadcast_to(q.reshape(qb, 1, D), (qb, K, D)).reshape(qb * K, D)
    prod = k * qrep
    scores = jnp.dot(prod, sum_ref[...],
                     preferred_element_type=jnp.float32) * (C ** -0.5)
    scores = scores.reshape(qb, K, H)
    mx = jnp.max(scores, axis=1, keepdims=True)
    e = jnp.exp(scores - mx)
    attn = (e / jnp.sum(e, axis=1, keepdims=True)).reshape(qb * K, H)
    attn_exp = jnp.dot(attn, exp_ref[...], preferred_element_type=jnp.float32)
    hid = jnp.sum((attn_exp * v).reshape(qb, K, D), axis=1)
    hid = jnp.dot(hid, wl_ref[...], preferred_element_type=jnp.float32) + bl_ref[...]
    x = _ln(hid + xin, g1_ref[...], bn1_ref[...])
    h = jnp.maximum(
        jnp.dot(x, we_ref[...], preferred_element_type=jnp.float32) + be_ref[...], 0.0
    )
    h = jnp.dot(h, ws_ref[...], preferred_element_type=jnp.float32) + bs_ref[...]
    out_ref[...] = _ln(x + h, g2_ref[...], bn2_ref[...])


def _attn_post(qstart, nq, kvg, x2d, wq_t, bq2, sum_mat, exp_mat,
               wl_t, bl2, g12, bn12, we_t, be2, ws_t, bs2, g22, bn22):
    base = qstart // QB
    grid = (nq // QB,)
    wspec = lambda shape: pl.BlockSpec(shape, lambda i: (0, 0))
    return pl.pallas_call(
        _attn_body,
        grid=grid,
        in_specs=[
            pl.BlockSpec((QB * K, D), lambda i: (i, 0)),
            pl.BlockSpec((QB, D), lambda i: (base + i, 0)),
            wspec((D, D)), wspec((1, D)),
            wspec((D, H)), wspec((H, D)),
            wspec((D, D)), wspec((1, D)), wspec((1, D)), wspec((1, D)),
            wspec((D, 2 * D)), wspec((1, 2 * D)),
            wspec((2 * D, D)), wspec((1, D)), wspec((1, D)), wspec((1, D)),
        ],
        out_specs=pl.BlockSpec((QB, D), lambda i: (i, 0)),
        out_shape=jax.ShapeDtypeStruct((nq, D), jnp.float32),
    )(kvg, x2d, wq_t, bq2, sum_mat, exp_mat,
      wl_t, bl2, g12, bn12, we_t, be2, ws_t, bs2, g22, bn22)


def kernel(input_states, memory_states, indices, Wq, bq, Wk, bk, Wv, bv,
           Wl, bl, g1, bn1, We, be, Ws, bs, g2, bn2):
    x2d = input_states.reshape(B * N, D)
    m2d = memory_states.reshape(B * M, D)
    wkv_t = jnp.concatenate([Wk.T, Wv.T], axis=1)
    bkv2 = jnp.concatenate([bk, bv]).reshape(1, 2 * D)

    head = jnp.arange(D, dtype=jnp.int32) // C
    sum_mat = (head[:, None] == jnp.arange(H, dtype=jnp.int32)[None, :]
               ).astype(jnp.float32)
    exp_mat = sum_mat.T

    kv_tables = [_proj_kv(m2d, wkv_t, bkv2, b) for b in range(B)]
    idx_flat = indices.reshape(NIDX)
    post_args = (
        Wq.T, bq.reshape(1, D), sum_mat, exp_mat,
        Wl.T, bl.reshape(1, D), g1.reshape(1, D), bn1.reshape(1, D),
        We.T, be.reshape(1, 2 * D), Ws.T, bs.reshape(1, D),
        g2.reshape(1, D), bn2.reshape(1, D),
    )
    outs = []
    off = 0
    for nidx_p in CHUNKS:
        idx_c = lax.slice_in_dim(idx_flat, off, off + nidx_p)
        kvg = _sc_gather(idx_c, kv_tables[off // (N * K)], nidx_p)
        outs.append(_attn_post(off // K, nidx_p // K, kvg, x2d, *post_args))
        off += nidx_p
    return jnp.concatenate(outs, axis=0).reshape(B, N, D)
